# baseline (speedup 1.0000x reference)
import functools

import jax
import jax.numpy as jnp
import numpy as np
from jax import lax
from jax.experimental import pallas as pl
from jax.experimental.pallas import tpu as pltpu
from jax.experimental.pallas import tpu_sc as plsc

_DIM_SIZE = 100000
_N_KEEP = 50000
_ROWS = 64

_L = 16
_NW = 32
_K = 2000
_CPR = _N_KEEP // _K
_NPAIR = _CPR


def _kept_indices():
    cpu = jax.local_devices(backend="cpu")[0]
    with jax.default_device(cpu):
        perm = jax.random.permutation(jax.random.key(42), _DIM_SIZE)
        return np.asarray(perm[:_N_KEEP], dtype=np.int32)


_IDX = _kept_indices()


@functools.lru_cache(maxsize=None)
def _build_kernel():
    mesh = plsc.VectorSubcoreMesh(core_axis_name="c", subcore_axis_name="s")

    @functools.partial(
        pl.kernel,
        mesh=mesh,
        out_type=jax.ShapeDtypeStruct((_ROWS, _N_KEEP), jnp.float32),
        scratch_types=[
            pltpu.VMEM((_DIM_SIZE,), jnp.float32),
            pltpu.VMEM((_K,), jnp.int32),
            pltpu.VMEM((_K,), jnp.int32),
            pltpu.VMEM((_K,), jnp.float32),
            pltpu.VMEM((_K,), jnp.float32),
            pltpu.SemaphoreType.DMA,
            pltpu.SemaphoreType.DMA,
            pltpu.SemaphoreType.DMA,
            pltpu.SemaphoreType.DMA,
            pltpu.SemaphoreType.DMA,
        ],
        compiler_params=pltpu.CompilerParams(
            use_tc_tiling_on_sc=False, needs_layout_passes=False
        ),
    )
    def _gather_kernel(
        x_hbm, idx_hbm, out_hbm,
        row_v, idx0_v, idx1_v, out0_v, out1_v,
        row_sem, idx0_sem, idx1_sem, out0_sem, out1_sem,
    ):
        wid = lax.axis_index("s") * 2 + lax.axis_index("c")
        r0 = wid * 2

        idx_v = [idx0_v, idx1_v]
        out_v = [out0_v, out1_v]
        idx_sem = [idx0_sem, idx1_sem]
        out_sem = [out0_sem, out1_sem]

        def gather_chunk(idx_b, out_b):
            @plsc.parallel_loop(0, _K // _L, unroll=125)
            def _(i):
                sl = pl.ds(i * _L, _L)
                out_b[sl] = plsc.load_gather(row_v, [idx_b[sl]])

        def wait_idx(k):
            pltpu.make_async_copy(
                idx_hbm.at[pl.ds(0, _K)], idx_v[k], idx_sem[k]
            ).wait()

        def wait_out(k):
            pltpu.make_async_copy(
                out_v[k], out_hbm.at[0, pl.ds(0, _K)], out_sem[k]
            ).wait()

        row_cp = pltpu.async_copy(x_hbm.at[r0], row_v, row_sem)
        pltpu.async_copy(idx_hbm.at[pl.ds(0, _K)], idx_v[0], idx_sem[0])
        pltpu.async_copy(idx_hbm.at[pl.ds(_K, _K)], idx_v[1], idx_sem[1])
        with jax.named_scope("row0_load_wait"):
            row_cp.wait()

        def sub_chunk(t, k):
            g = 2 * t + k
            loc = jnp.where(g >= _CPR, g - _CPR, g)
            r = jnp.where(g >= _CPR, r0 + 1, r0)
            j0 = loc * _K
            wait_idx(k)

            @pl.when(t > 0)
            def _():
                wait_out(k)

            gather_chunk(idx_v[k], out_v[k])
            pltpu.async_copy(out_v[k], out_hbm.at[r, pl.ds(j0, _K)], out_sem[k])

            @pl.when(t < _NPAIR - 1)
            def _():
                gn = g + 2
                j0n = jnp.where(gn >= _CPR, gn - _CPR, gn) * _K
                pltpu.async_copy(
                    idx_hbm.at[pl.ds(j0n, _K)], idx_v[k], idx_sem[k]
                )

        def pair_body(t, carry):
            sub_chunk(t, 0)

            @pl.when(t == (_CPR - 1) // 2)
            def _():
                cp = pltpu.async_copy(x_hbm.at[r0 + 1], row_v, row_sem)
                with jax.named_scope("row1_load_wait"):
                    cp.wait()

            sub_chunk(t, 1)
            return carry

        lax.fori_loop(0, _NPAIR, pair_body, 0)

        wait_out(0)
        wait_out(1)

    return _gather_kernel


@jax.jit
def kernel(x):
    return _build_kernel()(x, jnp.asarray(_IDX))


# speedup vs baseline: 1.2768x; 1.2768x over previous
import functools

import jax
import jax.numpy as jnp
import numpy as np
from jax import lax
from jax.experimental import pallas as pl
from jax.experimental.pallas import tpu as pltpu
from jax.experimental.pallas import tpu_sc as plsc

_DIM_SIZE = 100000
_N_KEEP = 50000
_ROWS = 64

_L = 16
_NW = 32
_K = 2000
_CPR = _N_KEEP // _K
_NPAIR = _CPR


def _kept_indices():
    cpu = jax.local_devices(backend="cpu")[0]
    with jax.default_device(cpu):
        perm = jax.random.permutation(jax.random.key(42), _DIM_SIZE)
        return np.asarray(perm[:_N_KEEP], dtype=np.int32)


_IDX = _kept_indices()


@functools.lru_cache(maxsize=None)
def _build_kernel():
    mesh = plsc.VectorSubcoreMesh(core_axis_name="c", subcore_axis_name="s")

    @functools.partial(
        pl.kernel,
        mesh=mesh,
        out_type=jax.ShapeDtypeStruct((_ROWS, _N_KEEP), jnp.float32),
        scratch_types=[
            pltpu.VMEM((_DIM_SIZE,), jnp.float32),
            pltpu.VMEM((_K,), jnp.int32),
            pltpu.VMEM((_K,), jnp.int32),
            pltpu.VMEM((_K,), jnp.float32),
            pltpu.VMEM((_K,), jnp.float32),
            pltpu.VMEM_SHARED((_N_KEEP,), jnp.int32),
            pltpu.SemaphoreType.DMA,
            pltpu.SemaphoreType.DMA,
            pltpu.SemaphoreType.DMA,
            pltpu.SemaphoreType.DMA,
            pltpu.SemaphoreType.DMA,
        ],
        compiler_params=pltpu.CompilerParams(
            use_tc_tiling_on_sc=False, needs_layout_passes=False
        ),
    )
    def _gather_kernel(
        x_hbm, idx_hbm, out_hbm,
        row_v, idx0_v, idx1_v, out0_v, out1_v, idx_sh,
        row_sem, idx0_sem, idx1_sem, out0_sem, out1_sem,
    ):
        wid = lax.axis_index("s") * 2 + lax.axis_index("c")
        r0 = wid * 2

        idx_v = [idx0_v, idx1_v]
        out_v = [out0_v, out1_v]
        idx_sem = [idx0_sem, idx1_sem]
        out_sem = [out0_sem, out1_sem]

        def gather_chunk(idx_b, out_b):
            @plsc.parallel_loop(0, _K // _L, unroll=25)
            def _(i):
                sl = pl.ds(i * _L, _L)
                out_b[sl] = plsc.load_gather(row_v, [idx_b[sl]])

        def wait_idx(k):
            pltpu.make_async_copy(
                idx_sh.at[pl.ds(0, _K)], idx_v[k], idx_sem[k]
            ).wait()

        def wait_out(k):
            pltpu.make_async_copy(
                out_v[k], out_hbm.at[0, pl.ds(0, _K)], out_sem[k]
            ).wait()

        row_cp = pltpu.async_copy(x_hbm.at[r0], row_v, row_sem)

        @pl.when(lax.axis_index("s") == 0)
        def _():
            pltpu.sync_copy(idx_hbm, idx_sh)

        plsc.subcore_barrier()
        pltpu.async_copy(idx_sh.at[pl.ds(0, _K)], idx_v[0], idx_sem[0])
        pltpu.async_copy(idx_sh.at[pl.ds(_K, _K)], idx_v[1], idx_sem[1])
        with jax.named_scope("row0_load_wait"):
            row_cp.wait()

        def sub_chunk(t, k):
            g = 2 * t + k
            loc = jnp.where(g >= _CPR, g - _CPR, g)
            r = jnp.where(g >= _CPR, r0 + 1, r0)
            j0 = loc * _K
            wait_idx(k)

            @pl.when(t > 0)
            def _():
                wait_out(k)

            gather_chunk(idx_v[k], out_v[k])
            pltpu.async_copy(out_v[k], out_hbm.at[r, pl.ds(j0, _K)], out_sem[k])

            @pl.when(t < _NPAIR - 1)
            def _():
                gn = g + 2
                j0n = jnp.where(gn >= _CPR, gn - _CPR, gn) * _K
                pltpu.async_copy(
                    idx_sh.at[pl.ds(j0n, _K)], idx_v[k], idx_sem[k]
                )

        def pair_body(t, carry):
            sub_chunk(t, 0)

            @pl.when(t == (_CPR - 1) // 2)
            def _():
                cp = pltpu.async_copy(x_hbm.at[r0 + 1], row_v, row_sem)
                with jax.named_scope("row1_load_wait"):
                    cp.wait()

            sub_chunk(t, 1)
            return carry

        lax.fori_loop(0, _NPAIR, pair_body, 0)

        wait_out(0)
        wait_out(1)

    return _gather_kernel


@jax.jit
def kernel(x):
    return _build_kernel()(x, jnp.asarray(_IDX))
